# baseline (speedup 1.0000x reference)
import functools

import jax
import jax.numpy as jnp
from jax import lax
from jax.experimental import pallas as pl
from jax.experimental.pallas import tpu as pltpu
from jax.experimental.pallas import tpu_sc as plsc

NC = 2
NS = 16
NW = NC * NS
K = 128
L = 16


@functools.partial(jax.jit, static_argnames=("n_pad", "w", "with_cnt"))
def _sc_scatter(table, srcs, dst2, zrow, zcnt, ones, *, n_pad, w, with_cnt):
    npass = len(srcs)
    tch, kk = dst2.shape
    if w == 64:
        gk = 4 if with_cnt else 5
    else:
        gk = 10
    ngroups = tch // gk
    base = ngroups // NW
    extra = ngroups % NW
    rpt = n_pad // NS

    mesh = plsc.VectorSubcoreMesh(core_axis_name="c", subcore_axis_name="s")
    out_type = [jax.ShapeDtypeStruct((n_pad, 128), jnp.float32)
                for _ in range(npass)]
    scratch = [
        pltpu.VMEM((2, gk, kk), jnp.int32),
        pltpu.VMEM((2, gk, kk), jnp.int32),
        pltpu.VMEM((2, gk, kk, w), jnp.float32),
        pltpu.VMEM_SHARED((n_pad, w), jnp.float32),
        pltpu.SemaphoreType.DMA,
        pltpu.SemaphoreType.DMA,
    ]
    if with_cnt:
        out_type.append(jax.ShapeDtypeStruct((n_pad, 128), jnp.float32))
        scratch += [
            pltpu.VMEM((kk, 16), jnp.float32),
            pltpu.VMEM_SHARED((n_pad, 16), jnp.float32),
        ]

    @functools.partial(
        pl.kernel, mesh=mesh, out_type=out_type, scratch_types=scratch,
        compiler_params=pltpu.CompilerParams(use_tc_tiling_on_sc=False))
    def body(t_hbm, *rest):
        src_hbms = rest[:npass]
        dst_hbm, zrow_hbm, zcnt_hbm, ones_hbm = rest[npass:npass + 4]
        refs = rest[npass + 4:]
        if with_cnt:
            outs = refs[:npass]
            (c_out, src_v, dst_v, rows_v, s_sh, gsem, ssem,
             ones_v, c_sh) = refs[npass:]
        else:
            outs = refs[:npass]
            (src_v, dst_v, rows_v, s_sh, gsem, ssem) = refs[npass:]
        c = lax.axis_index("c")
        s = lax.axis_index("s")
        t = c * NS + s

        ng = base + jnp.where(t < extra, 1, 0)
        g0 = t * base + jnp.minimum(t, extra)

        if with_cnt:
            pltpu.sync_copy(ones_hbm, ones_v)

        def one_pass(src_hbm, out_hbm, do_cnt):
            pltpu.sync_copy(zrow_hbm, s_sh.at[pl.ds(s * rpt, rpt)])
            if do_cnt:
                pltpu.sync_copy(zcnt_hbm, c_sh.at[pl.ds(s * rpt, rpt)])
            plsc.subcore_barrier()

            def load_and_fire(b, g):
                chunk0 = (g0 + g) * gk
                pltpu.sync_copy(src_hbm.at[pl.ds(chunk0, gk)], src_v.at[b])
                pltpu.sync_copy(dst_hbm.at[pl.ds(chunk0, gk)], dst_v.at[b])
                for j in range(gk):
                    pltpu.async_copy(t_hbm.at[src_v.at[b, j]],
                                     rows_v.at[b, j], gsem)

            def drain_scatters(b):
                for j in range(gk):
                    pltpu.make_async_copy(rows_v.at[b, j],
                                          s_sh.at[dst_v.at[b, j]],
                                          ssem).wait()
                    if do_cnt:
                        pltpu.make_async_copy(ones_v,
                                              c_sh.at[dst_v.at[b, j]],
                                              ssem).wait()

            @pl.when(ng > 0)
            def _():
                load_and_fire(0, 0)

            def step(g, carry):
                b = lax.rem(g, 2)
                for j in range(gk):
                    pltpu.make_async_copy(t_hbm.at[src_v.at[b, j]],
                                          rows_v.at[b, j], gsem).wait()
                for j in range(gk):
                    pltpu.async_copy(rows_v.at[b, j], s_sh.at[dst_v.at[b, j]],
                                     ssem, add=True)
                    if do_cnt:
                        pltpu.async_copy(ones_v, c_sh.at[dst_v.at[b, j]],
                                         ssem, add=True)

                @pl.when(g + 1 < ng)
                def _():
                    @pl.when(g >= 1)
                    def _():
                        drain_scatters(1 - b)
                    load_and_fire(1 - b, g + 1)
                return carry

            lax.fori_loop(0, ng, step, 0)

            @pl.when(ng >= 2)
            def _():
                drain_scatters(lax.rem(ng, 2))

            @pl.when(ng >= 1)
            def _():
                drain_scatters(lax.rem(ng + 1, 2))
            plsc.subcore_barrier()
            pltpu.sync_copy(s_sh.at[pl.ds(s * rpt, rpt)],
                            out_hbm.at[pl.ds(s * rpt, rpt), pl.ds(w * c, w)])
            if do_cnt:
                pltpu.sync_copy(
                    c_sh.at[pl.ds(s * rpt, rpt)],
                    c_out.at[pl.ds(s * rpt, rpt), pl.ds(16 * c, 16)])

        for p in range(npass):
            one_pass(src_hbms[p], outs[p], with_cnt and p == 0)

    return body(table, *srcs, dst2, zrow, zcnt, ones)


def _bn_relu(a, g, be):
    mu = jnp.mean(a, axis=0, keepdims=True)
    var = jnp.mean((a - mu) * (a - mu), axis=0, keepdims=True)
    h = (a - mu) * lax.rsqrt(var + 1e-5) * g + be
    return jnp.maximum(h, 0.0)


def _tc_pre(x, wl, wr):
    def body(x_ref, wl_ref, wr_ref, t_ref, q_ref):
        xv = x_ref[...]
        t_ref[...] = jnp.dot(xv, wl_ref[...],
                             preferred_element_type=jnp.float32)
        q_ref[...] = jnp.dot(xv, wr_ref[...],
                             preferred_element_type=jnp.float32)
    sh = jax.ShapeDtypeStruct((x.shape[0], wl.shape[1]), jnp.float32)
    return pl.pallas_call(body, out_shape=[sh, sh])(x, wl, wr)


def _tc_mid1(sa, sb, cnt, q1, b, g, be, wl, wr, *, n):
    def body(sa_ref, sb_ref, c_ref, q_ref, b_ref, g_ref, be_ref,
             wl_ref, wr_ref, o_ref):
        s_lo = sa_ref[0:n, 0:64] + sa_ref[0:n, 64:128]
        s_hi = sb_ref[0:n, 0:64] + sb_ref[0:n, 64:128]
        ssum = jnp.concatenate([s_lo, s_hi], axis=1)
        cv = c_ref[0:n, 0:1] + c_ref[0:n, 16:17]
        a = ssum / jnp.maximum(cv, 1.0) + b_ref[...] + q_ref[...]
        h = _bn_relu(a, g_ref[...], be_ref[...])
        o_ref[...] = jnp.concatenate(
            [jnp.dot(h, wl_ref[...], preferred_element_type=jnp.float32),
             jnp.dot(h, wr_ref[...], preferred_element_type=jnp.float32)],
            axis=1)

    return pl.pallas_call(
        body, out_shape=jax.ShapeDtypeStruct((n, 128), jnp.float32),
    )(sa, sb, cnt, q1, b.reshape(1, -1), g.reshape(1, -1), be.reshape(1, -1),
      wl, wr)


def _tc_mid2(s2, cnt, t2, b, g, be, wl, wr, *, n):
    def body(s_ref, c_ref, t_ref, b_ref, g_ref, be_ref, wl_ref, wr_ref,
             o_ref):
        ssum = s_ref[0:n, 0:64] + s_ref[0:n, 64:128]
        cv = c_ref[0:n, 0:1] + c_ref[0:n, 16:17]
        a = ssum / jnp.maximum(cv, 1.0) + b_ref[...] + t_ref[0:n, 64:128]
        h = _bn_relu(a, g_ref[...], be_ref[...])
        p3 = jnp.dot(h, wl_ref[...], preferred_element_type=jnp.float32)
        q3 = jnp.dot(h, wr_ref[...], preferred_element_type=jnp.float32)
        o_ref[:, 0:64] = jnp.concatenate([p3, q3], axis=1)

    return pl.pallas_call(
        body, out_shape=jax.ShapeDtypeStruct((n, 128), jnp.float32),
    )(s2, cnt, t2, b.reshape(1, -1), g.reshape(1, -1), be.reshape(1, -1),
      wl, wr)


def _tc_final(s3, cnt, t3, b, g, be, batch2, f1w, f1b, f2w, f2b, f3w, f3b,
              *, n, g_groups):
    def body(s_ref, c_ref, t_ref, b_ref, g_ref, be_ref, batch_ref,
             f1w_ref, f1b_ref, f2w_ref, f2b_ref, f3w_ref, f3b_ref, o_ref):
        ssum = s_ref[0:n, 0:32] + s_ref[0:n, 32:64]
        cv = c_ref[0:n, 0:1] + c_ref[0:n, 16:17]
        a = ssum / jnp.maximum(cv, 1.0) + b_ref[...] + t_ref[0:n, 32:64]
        h = _bn_relu(a, g_ref[...], be_ref[...])

        gid = lax.broadcasted_iota(jnp.int32, (g_groups, n), 0)
        onehot = (gid == batch_ref[...]).astype(jnp.float32)
        gsum = jnp.dot(onehot, h, preferred_element_type=jnp.float32)
        gcnt = jnp.sum(onehot, axis=1, keepdims=True)
        hp = gsum / jnp.maximum(gcnt, 1.0)

        hp = jnp.maximum(jnp.dot(hp, f1w_ref[...],
                                 preferred_element_type=jnp.float32)
                         + f1b_ref[...], 0.0)
        hp = jnp.maximum(jnp.dot(hp, f2w_ref[...],
                                 preferred_element_type=jnp.float32)
                         + f2b_ref[...], 0.0)
        o_ref[...] = jnp.dot(hp, f3w_ref[...],
                             preferred_element_type=jnp.float32) + f3b_ref[...]

    return pl.pallas_call(
        body,
        out_shape=jax.ShapeDtypeStruct((g_groups, f3w.shape[1]), jnp.float32),
    )(s3, cnt, t3, b.reshape(1, -1), g.reshape(1, -1), be.reshape(1, -1),
      batch2, f1w, f1b.reshape(1, -1), f2w, f2b.reshape(1, -1), f3w,
      f3b.reshape(1, -1))


def kernel(x, edge_index, batch, W1l, b1, W1r, g1, be1, W2l, b2, W2r, g2, be2,
           W3l, b3, W3r, g3, be3, f1W, f1b, f2W, f2b, f3W, f3b):
    n, d = x.shape
    n_pad = ((n + NS * 8 - 1) // (NS * 8)) * (NS * 8)
    rpt = n_pad // NS
    e = edge_index.shape[1]
    tch = e // K
    src_a = (edge_index[0] * 2).reshape(tch, K)
    src_b = src_a + 1
    src_4 = src_a * 2
    dst2 = edge_index[1].reshape(tch, K)
    g_groups = 64

    zrow64 = jnp.zeros((rpt, 64), jnp.float32)
    zrow32 = jnp.zeros((rpt, 32), jnp.float32)
    zcnt = jnp.zeros((rpt, 16), jnp.float32)
    ones = jnp.ones((K, 16), jnp.float32)
    batch2 = batch.reshape(1, n)

    t1, q1 = _tc_pre(x, W1l, W1r)
    sa1, sb1, c1 = _sc_scatter(t1.reshape(2 * n, 64), (src_a, src_b), dst2,
                               zrow64, zcnt, ones,
                               n_pad=n_pad, w=64, with_cnt=True)
    t2 = _tc_mid1(sa1, sb1, c1, q1, b1, g1, be1, W2l, W2r, n=n)
    (s2,) = _sc_scatter(t2.reshape(2 * n, 64), (src_a,), dst2,
                        zrow64, zcnt, ones, n_pad=n_pad, w=64, with_cnt=False)
    t3 = _tc_mid2(s2, c1, t2, b2, g2, be2, W3l, W3r, n=n)
    (s3,) = _sc_scatter(t3.reshape(4 * n, 32), (src_4,), dst2,
                        zrow32, zcnt, ones, n_pad=n_pad, w=32, with_cnt=False)
    return _tc_final(s3, c1, t3, b3, g3, be3, batch2,
                     f1W, f1b, f2W, f2b, f3W, f3b,
                     n=n, g_groups=g_groups)


# speedup vs baseline: 1.0498x; 1.0498x over previous
import functools

import jax
import jax.numpy as jnp
from jax import lax
from jax.experimental import pallas as pl
from jax.experimental.pallas import tpu as pltpu
from jax.experimental.pallas import tpu_sc as plsc

NC = 2
NS = 16
NW = NC * NS
K = 128
L = 16


@functools.partial(jax.jit, static_argnames=("n_pad", "w", "with_cnt"))
def _sc_scatter(table, srcs, dst2, zrow, zcnt, ones, *, n_pad, w, with_cnt):
    npass = len(srcs)
    tch, kk = dst2.shape
    if w == 64:
        gk = 4 if with_cnt else 5
    else:
        gk = 10
    ngroups = tch // gk
    base = ngroups // NW
    extra = ngroups % NW
    rpt = n_pad // NS

    mesh = plsc.VectorSubcoreMesh(core_axis_name="c", subcore_axis_name="s")
    out_type = [jax.ShapeDtypeStruct((n_pad, 128), jnp.float32)
                for _ in range(npass)]
    scratch = [
        pltpu.VMEM((2, gk, kk), jnp.int32),
        pltpu.VMEM((2, gk, kk), jnp.int32),
        pltpu.VMEM((2, gk, kk, w), jnp.float32),
        pltpu.VMEM_SHARED((n_pad, w), jnp.float32),
        pltpu.SemaphoreType.DMA,
        pltpu.SemaphoreType.DMA,
    ]
    if with_cnt:
        out_type.append(jax.ShapeDtypeStruct((n_pad, 128), jnp.float32))
        scratch += [
            pltpu.VMEM((kk, 16), jnp.float32),
            pltpu.VMEM_SHARED((n_pad, 16), jnp.float32),
        ]

    @functools.partial(
        pl.kernel, mesh=mesh, out_type=out_type, scratch_types=scratch,
        compiler_params=pltpu.CompilerParams(use_tc_tiling_on_sc=False))
    def body(t_hbm, *rest):
        src_hbms = rest[:npass]
        dst_hbm, zrow_hbm, zcnt_hbm, ones_hbm = rest[npass:npass + 4]
        refs = rest[npass + 4:]
        if with_cnt:
            outs = refs[:npass]
            (c_out, src_v, dst_v, rows_v, s_sh, gsem, ssem,
             ones_v, c_sh) = refs[npass:]
        else:
            outs = refs[:npass]
            (src_v, dst_v, rows_v, s_sh, gsem, ssem) = refs[npass:]
        c = lax.axis_index("c")
        s = lax.axis_index("s")
        t = c * NS + s

        ng = base + jnp.where(t < extra, 1, 0)
        g0 = t * base + jnp.minimum(t, extra)

        if with_cnt:
            pltpu.sync_copy(ones_hbm, ones_v)

        def one_pass(src_hbm, out_hbm, do_cnt):
            pltpu.sync_copy(zrow_hbm, s_sh.at[pl.ds(s * rpt, rpt)])
            if do_cnt:
                pltpu.sync_copy(zcnt_hbm, c_sh.at[pl.ds(s * rpt, rpt)])
            plsc.subcore_barrier()

            def load_and_fire(b, g):
                chunk0 = (g0 + g) * gk
                pltpu.sync_copy(src_hbm.at[pl.ds(chunk0, gk)], src_v.at[b])
                pltpu.sync_copy(dst_hbm.at[pl.ds(chunk0, gk)], dst_v.at[b])
                for j in range(gk):
                    pltpu.async_copy(t_hbm.at[src_v.at[b, j]],
                                     rows_v.at[b, j], gsem)

            @pl.when(ng > 0)
            def _():
                load_and_fire(0, 0)

            def step(g, carry):
                b = lax.rem(g, 2)

                @pl.when(g + 1 < ng)
                def _():
                    load_and_fire(1 - b, g + 1)

                for j in range(gk):
                    pltpu.make_async_copy(t_hbm.at[src_v.at[b, j]],
                                          rows_v.at[b, j], gsem).wait()
                for j in range(gk):
                    pltpu.sync_copy(rows_v.at[b, j], s_sh.at[dst_v.at[b, j]],
                                    add=True)
                    if do_cnt:
                        pltpu.sync_copy(ones_v, c_sh.at[dst_v.at[b, j]],
                                        add=True)
                return carry

            lax.fori_loop(0, ng, step, 0)
            plsc.subcore_barrier()
            pltpu.sync_copy(s_sh.at[pl.ds(s * rpt, rpt)],
                            out_hbm.at[pl.ds(s * rpt, rpt), pl.ds(w * c, w)])
            if do_cnt:
                pltpu.sync_copy(
                    c_sh.at[pl.ds(s * rpt, rpt)],
                    c_out.at[pl.ds(s * rpt, rpt), pl.ds(16 * c, 16)])

        for p in range(npass):
            one_pass(src_hbms[p], outs[p], with_cnt and p == 0)

    return body(table, *srcs, dst2, zrow, zcnt, ones)


def _bn_relu(a, g, be):
    mu = jnp.mean(a, axis=0, keepdims=True)
    var = jnp.mean((a - mu) * (a - mu), axis=0, keepdims=True)
    h = (a - mu) * lax.rsqrt(var + 1e-5) * g + be
    return jnp.maximum(h, 0.0)


def _tc_pre(x, wl, wr):
    def body(x_ref, wl_ref, wr_ref, t_ref, q_ref):
        xv = x_ref[...]
        t_ref[...] = jnp.dot(xv, wl_ref[...],
                             preferred_element_type=jnp.float32)
        q_ref[...] = jnp.dot(xv, wr_ref[...],
                             preferred_element_type=jnp.float32)
    sh = jax.ShapeDtypeStruct((x.shape[0], wl.shape[1]), jnp.float32)
    return pl.pallas_call(body, out_shape=[sh, sh])(x, wl, wr)


def _tc_mid1(sa, sb, cnt, q1, b, g, be, wl, wr, *, n):
    def body(sa_ref, sb_ref, c_ref, q_ref, b_ref, g_ref, be_ref,
             wl_ref, wr_ref, o_ref):
        s_lo = sa_ref[0:n, 0:64] + sa_ref[0:n, 64:128]
        s_hi = sb_ref[0:n, 0:64] + sb_ref[0:n, 64:128]
        ssum = jnp.concatenate([s_lo, s_hi], axis=1)
        cv = c_ref[0:n, 0:1] + c_ref[0:n, 16:17]
        a = ssum / jnp.maximum(cv, 1.0) + b_ref[...] + q_ref[...]
        h = _bn_relu(a, g_ref[...], be_ref[...])
        o_ref[...] = jnp.concatenate(
            [jnp.dot(h, wl_ref[...], preferred_element_type=jnp.float32),
             jnp.dot(h, wr_ref[...], preferred_element_type=jnp.float32)],
            axis=1)

    return pl.pallas_call(
        body, out_shape=jax.ShapeDtypeStruct((n, 128), jnp.float32),
    )(sa, sb, cnt, q1, b.reshape(1, -1), g.reshape(1, -1), be.reshape(1, -1),
      wl, wr)


def _tc_mid2(s2, cnt, t2, b, g, be, wl, wr, *, n):
    def body(s_ref, c_ref, t_ref, b_ref, g_ref, be_ref, wl_ref, wr_ref,
             o_ref):
        ssum = s_ref[0:n, 0:64] + s_ref[0:n, 64:128]
        cv = c_ref[0:n, 0:1] + c_ref[0:n, 16:17]
        a = ssum / jnp.maximum(cv, 1.0) + b_ref[...] + t_ref[0:n, 64:128]
        h = _bn_relu(a, g_ref[...], be_ref[...])
        p3 = jnp.dot(h, wl_ref[...], preferred_element_type=jnp.float32)
        q3 = jnp.dot(h, wr_ref[...], preferred_element_type=jnp.float32)
        o_ref[:, 0:64] = jnp.concatenate([p3, q3], axis=1)

    return pl.pallas_call(
        body, out_shape=jax.ShapeDtypeStruct((n, 128), jnp.float32),
    )(s2, cnt, t2, b.reshape(1, -1), g.reshape(1, -1), be.reshape(1, -1),
      wl, wr)


def _tc_final(s3, cnt, t3, b, g, be, batch2, f1w, f1b, f2w, f2b, f3w, f3b,
              *, n, g_groups):
    def body(s_ref, c_ref, t_ref, b_ref, g_ref, be_ref, batch_ref,
             f1w_ref, f1b_ref, f2w_ref, f2b_ref, f3w_ref, f3b_ref, o_ref):
        ssum = s_ref[0:n, 0:32] + s_ref[0:n, 32:64]
        cv = c_ref[0:n, 0:1] + c_ref[0:n, 16:17]
        a = ssum / jnp.maximum(cv, 1.0) + b_ref[...] + t_ref[0:n, 32:64]
        h = _bn_relu(a, g_ref[...], be_ref[...])

        gid = lax.broadcasted_iota(jnp.int32, (g_groups, n), 0)
        onehot = (gid == batch_ref[...]).astype(jnp.float32)
        gsum = jnp.dot(onehot, h, preferred_element_type=jnp.float32)
        gcnt = jnp.sum(onehot, axis=1, keepdims=True)
        hp = gsum / jnp.maximum(gcnt, 1.0)

        hp = jnp.maximum(jnp.dot(hp, f1w_ref[...],
                                 preferred_element_type=jnp.float32)
                         + f1b_ref[...], 0.0)
        hp = jnp.maximum(jnp.dot(hp, f2w_ref[...],
                                 preferred_element_type=jnp.float32)
                         + f2b_ref[...], 0.0)
        o_ref[...] = jnp.dot(hp, f3w_ref[...],
                             preferred_element_type=jnp.float32) + f3b_ref[...]

    return pl.pallas_call(
        body,
        out_shape=jax.ShapeDtypeStruct((g_groups, f3w.shape[1]), jnp.float32),
    )(s3, cnt, t3, b.reshape(1, -1), g.reshape(1, -1), be.reshape(1, -1),
      batch2, f1w, f1b.reshape(1, -1), f2w, f2b.reshape(1, -1), f3w,
      f3b.reshape(1, -1))


def kernel(x, edge_index, batch, W1l, b1, W1r, g1, be1, W2l, b2, W2r, g2, be2,
           W3l, b3, W3r, g3, be3, f1W, f1b, f2W, f2b, f3W, f3b):
    n, d = x.shape
    n_pad = ((n + NS * 8 - 1) // (NS * 8)) * (NS * 8)
    rpt = n_pad // NS
    e = edge_index.shape[1]
    tch = e // K
    src_a = (edge_index[0] * 2).reshape(tch, K)
    src_b = src_a + 1
    src_4 = src_a * 2
    dst2 = edge_index[1].reshape(tch, K)
    g_groups = 64

    zrow64 = jnp.zeros((rpt, 64), jnp.float32)
    zrow32 = jnp.zeros((rpt, 32), jnp.float32)
    zcnt = jnp.zeros((rpt, 16), jnp.float32)
    ones = jnp.ones((K, 16), jnp.float32)
    batch2 = batch.reshape(1, n)

    t1, q1 = _tc_pre(x, W1l, W1r)
    sa1, sb1, c1 = _sc_scatter(t1.reshape(2 * n, 64), (src_a, src_b), dst2,
                               zrow64, zcnt, ones,
                               n_pad=n_pad, w=64, with_cnt=True)
    t2 = _tc_mid1(sa1, sb1, c1, q1, b1, g1, be1, W2l, W2r, n=n)
    (s2,) = _sc_scatter(t2.reshape(2 * n, 64), (src_a,), dst2,
                        zrow64, zcnt, ones, n_pad=n_pad, w=64, with_cnt=False)
    t3 = _tc_mid2(s2, c1, t2, b2, g2, be2, W3l, W3r, n=n)
    (s3,) = _sc_scatter(t3.reshape(4 * n, 32), (src_4,), dst2,
                        zrow32, zcnt, ones, n_pad=n_pad, w=32, with_cnt=False)
    return _tc_final(s3, c1, t3, b3, g3, be3, batch2,
                     f1W, f1b, f2W, f2b, f3W, f3b,
                     n=n, g_groups=g_groups)


# speedup vs baseline: 1.0502x; 1.0004x over previous
import functools

import jax
import jax.numpy as jnp
from jax import lax
from jax.experimental import pallas as pl
from jax.experimental.pallas import tpu as pltpu
from jax.experimental.pallas import tpu_sc as plsc

NC = 2
NS = 16
NW = NC * NS
K = 128
L = 16


@functools.partial(jax.jit, static_argnames=("n_pad", "w", "with_cnt"))
def _sc_scatter(table, srcs, dst2, zrow, zcnt, ones, *, n_pad, w, with_cnt):
    npass = len(srcs)
    tch, kk = dst2.shape
    if w == 64:
        gk = 4 if with_cnt else 5
    else:
        gk = 10
    ngroups = tch // gk
    base = ngroups // NW
    extra = ngroups % NW
    rpt = n_pad // NS

    mesh = plsc.VectorSubcoreMesh(core_axis_name="c", subcore_axis_name="s")
    out_type = [jax.ShapeDtypeStruct((n_pad, 128), jnp.float32)
                for _ in range(npass)]
    scratch = [
        pltpu.VMEM((2, gk, kk), jnp.int32),
        pltpu.VMEM((2, gk, kk), jnp.int32),
        pltpu.VMEM((2, gk, kk, w), jnp.float32),
        pltpu.VMEM_SHARED((n_pad, w), jnp.float32),
        pltpu.SemaphoreType.DMA,
    ]
    if with_cnt:
        out_type.append(jax.ShapeDtypeStruct((n_pad, 128), jnp.float32))
        scratch += [
            pltpu.VMEM((kk, 16), jnp.float32),
            pltpu.VMEM_SHARED((n_pad, 16), jnp.float32),
        ]

    @functools.partial(
        pl.kernel, mesh=mesh, out_type=out_type, scratch_types=scratch,
        compiler_params=pltpu.CompilerParams(use_tc_tiling_on_sc=False))
    def body(t_hbm, *rest):
        src_hbms = rest[:npass]
        dst_hbm, zrow_hbm, zcnt_hbm, ones_hbm = rest[npass:npass + 4]
        refs = rest[npass + 4:]
        if with_cnt:
            outs = refs[:npass]
            (c_out, src_v, dst_v, rows_v, s_sh, gsem,
             ones_v, c_sh) = refs[npass:]
        else:
            outs = refs[:npass]
            (src_v, dst_v, rows_v, s_sh, gsem) = refs[npass:]
        c = lax.axis_index("c")
        s = lax.axis_index("s")
        t = c * NS + s

        ng = base + jnp.where(t < extra, 1, 0)
        g0 = t * base + jnp.minimum(t, extra)

        if with_cnt:
            pltpu.sync_copy(ones_hbm, ones_v)

        def one_pass(src_hbm, out_hbm, do_cnt):
            pltpu.sync_copy(zrow_hbm, s_sh.at[pl.ds(s * rpt, rpt)])
            if do_cnt:
                pltpu.sync_copy(zcnt_hbm, c_sh.at[pl.ds(s * rpt, rpt)])
            plsc.subcore_barrier()

            def load_and_fire(b, g):
                chunk0 = (g0 + g) * gk
                pltpu.sync_copy(src_hbm.at[pl.ds(chunk0, gk)], src_v.at[b])
                pltpu.sync_copy(dst_hbm.at[pl.ds(chunk0, gk)], dst_v.at[b])
                for j in range(gk):
                    pltpu.async_copy(t_hbm.at[src_v.at[b, j]],
                                     rows_v.at[b, j], gsem)

            @pl.when(ng > 0)
            def _():
                load_and_fire(0, 0)

            def step(g, carry):
                b = lax.rem(g, 2)

                @pl.when(g + 1 < ng)
                def _():
                    load_and_fire(1 - b, g + 1)

                for j in range(gk):
                    pltpu.make_async_copy(t_hbm.at[src_v.at[b, j]],
                                          rows_v.at[b, j], gsem).wait()
                for j in range(gk):
                    pltpu.sync_copy(rows_v.at[b, j], s_sh.at[dst_v.at[b, j]],
                                    add=True)
                    if do_cnt:
                        pltpu.sync_copy(ones_v, c_sh.at[dst_v.at[b, j]],
                                        add=True)
                return carry

            lax.fori_loop(0, ng, step, 0)
            plsc.subcore_barrier()
            pltpu.sync_copy(s_sh.at[pl.ds(s * rpt, rpt)],
                            out_hbm.at[pl.ds(s * rpt, rpt), pl.ds(w * c, w)])
            if do_cnt:
                pltpu.sync_copy(
                    c_sh.at[pl.ds(s * rpt, rpt)],
                    c_out.at[pl.ds(s * rpt, rpt), pl.ds(16 * c, 16)])

        for p in range(npass):
            one_pass(src_hbms[p], outs[p], with_cnt and p == 0)

    return body(table, *srcs, dst2, zrow, zcnt, ones)


def _bn_relu(a, g, be):
    mu = jnp.mean(a, axis=0, keepdims=True)
    var = jnp.mean((a - mu) * (a - mu), axis=0, keepdims=True)
    h = (a - mu) * lax.rsqrt(var + 1e-5) * g + be
    return jnp.maximum(h, 0.0)


def _tc_pre(x, wl, wr):
    def body(x_ref, wl_ref, wr_ref, t_ref, q_ref):
        xv = x_ref[...]
        t_ref[...] = jnp.dot(xv, wl_ref[...],
                             preferred_element_type=jnp.float32)
        q_ref[...] = jnp.dot(xv, wr_ref[...],
                             preferred_element_type=jnp.float32)
    sh = jax.ShapeDtypeStruct((x.shape[0], wl.shape[1]), jnp.float32)
    return pl.pallas_call(body, out_shape=[sh, sh])(x, wl, wr)


def _tc_mid1(sa, sb, cnt, q1, b, g, be, wl, wr, *, n):
    def body(sa_ref, sb_ref, c_ref, q_ref, b_ref, g_ref, be_ref,
             wl_ref, wr_ref, o_ref):
        s_lo = sa_ref[0:n, 0:64] + sa_ref[0:n, 64:128]
        s_hi = sb_ref[0:n, 0:64] + sb_ref[0:n, 64:128]
        ssum = jnp.concatenate([s_lo, s_hi], axis=1)
        cv = c_ref[0:n, 0:1] + c_ref[0:n, 16:17]
        a = ssum / jnp.maximum(cv, 1.0) + b_ref[...] + q_ref[...]
        h = _bn_relu(a, g_ref[...], be_ref[...])
        o_ref[...] = jnp.concatenate(
            [jnp.dot(h, wl_ref[...], preferred_element_type=jnp.float32),
             jnp.dot(h, wr_ref[...], preferred_element_type=jnp.float32)],
            axis=1)

    return pl.pallas_call(
        body, out_shape=jax.ShapeDtypeStruct((n, 128), jnp.float32),
    )(sa, sb, cnt, q1, b.reshape(1, -1), g.reshape(1, -1), be.reshape(1, -1),
      wl, wr)


def _tc_mid2(s2, cnt, t2, b, g, be, wl, wr, *, n):
    def body(s_ref, c_ref, t_ref, b_ref, g_ref, be_ref, wl_ref, wr_ref,
             o_ref):
        ssum = s_ref[0:n, 0:64] + s_ref[0:n, 64:128]
        cv = c_ref[0:n, 0:1] + c_ref[0:n, 16:17]
        a = ssum / jnp.maximum(cv, 1.0) + b_ref[...] + t_ref[0:n, 64:128]
        h = _bn_relu(a, g_ref[...], be_ref[...])
        p3 = jnp.dot(h, wl_ref[...], preferred_element_type=jnp.float32)
        q3 = jnp.dot(h, wr_ref[...], preferred_element_type=jnp.float32)
        o_ref[:, 0:64] = jnp.concatenate([p3, q3], axis=1)

    return pl.pallas_call(
        body, out_shape=jax.ShapeDtypeStruct((n, 128), jnp.float32),
    )(s2, cnt, t2, b.reshape(1, -1), g.reshape(1, -1), be.reshape(1, -1),
      wl, wr)


def _tc_final(s3, cnt, t3, b, g, be, batch2, f1w, f1b, f2w, f2b, f3w, f3b,
              *, n, g_groups):
    def body(s_ref, c_ref, t_ref, b_ref, g_ref, be_ref, batch_ref,
             f1w_ref, f1b_ref, f2w_ref, f2b_ref, f3w_ref, f3b_ref, o_ref):
        ssum = s_ref[0:n, 0:32] + s_ref[0:n, 32:64]
        cv = c_ref[0:n, 0:1] + c_ref[0:n, 16:17]
        a = ssum / jnp.maximum(cv, 1.0) + b_ref[...] + t_ref[0:n, 32:64]
        h = _bn_relu(a, g_ref[...], be_ref[...])

        gid = lax.broadcasted_iota(jnp.int32, (g_groups, n), 0)
        onehot = (gid == batch_ref[...]).astype(jnp.float32)
        gsum = jnp.dot(onehot, h, preferred_element_type=jnp.float32)
        gcnt = jnp.sum(onehot, axis=1, keepdims=True)
        hp = gsum / jnp.maximum(gcnt, 1.0)

        hp = jnp.maximum(jnp.dot(hp, f1w_ref[...],
                                 preferred_element_type=jnp.float32)
                         + f1b_ref[...], 0.0)
        hp = jnp.maximum(jnp.dot(hp, f2w_ref[...],
                                 preferred_element_type=jnp.float32)
                         + f2b_ref[...], 0.0)
        o_ref[...] = jnp.dot(hp, f3w_ref[...],
                             preferred_element_type=jnp.float32) + f3b_ref[...]

    return pl.pallas_call(
        body,
        out_shape=jax.ShapeDtypeStruct((g_groups, f3w.shape[1]), jnp.float32),
    )(s3, cnt, t3, b.reshape(1, -1), g.reshape(1, -1), be.reshape(1, -1),
      batch2, f1w, f1b.reshape(1, -1), f2w, f2b.reshape(1, -1), f3w,
      f3b.reshape(1, -1))


def kernel(x, edge_index, batch, W1l, b1, W1r, g1, be1, W2l, b2, W2r, g2, be2,
           W3l, b3, W3r, g3, be3, f1W, f1b, f2W, f2b, f3W, f3b):
    n, d = x.shape
    n_pad = ((n + NS * 8 - 1) // (NS * 8)) * (NS * 8)
    rpt = n_pad // NS
    e = edge_index.shape[1]
    tch = e // K
    src_a = (edge_index[0] * 2).reshape(tch, K)
    src_b = src_a + 1
    src_4 = src_a * 2
    dst2 = edge_index[1].reshape(tch, K)
    g_groups = 64

    zrow64 = jnp.zeros((rpt, 64), jnp.float32)
    zrow32 = jnp.zeros((rpt, 32), jnp.float32)
    zcnt = jnp.zeros((rpt, 16), jnp.float32)
    ones = jnp.ones((K, 16), jnp.float32)
    batch2 = batch.reshape(1, n)

    t1, q1 = _tc_pre(x, W1l, W1r)
    sa1, sb1, c1 = _sc_scatter(t1.reshape(2 * n, 64), (src_a, src_b), dst2,
                               zrow64, zcnt, ones,
                               n_pad=n_pad, w=64, with_cnt=True)
    t2 = _tc_mid1(sa1, sb1, c1, q1, b1, g1, be1, W2l, W2r, n=n)
    (s2,) = _sc_scatter(t2.reshape(2 * n, 64), (src_a,), dst2,
                        zrow64, zcnt, ones, n_pad=n_pad, w=64, with_cnt=False)
    t3 = _tc_mid2(s2, c1, t2, b2, g2, be2, W3l, W3r, n=n)
    (s3,) = _sc_scatter(t3.reshape(4 * n, 32), (src_4,), dst2,
                        zrow32, zcnt, ones, n_pad=n_pad, w=32, with_cnt=False)
    return _tc_final(s3, c1, t3, b3, g3, be3, batch2,
                     f1W, f1b, f2W, f2b, f3W, f3b,
                     n=n, g_groups=g_groups)
